# baseline (speedup 1.0000x reference)
import numpy as np
import jax
import jax.numpy as jnp
from jax import lax
from jax.experimental import pallas as pl
from jax.experimental.pallas import tpu as pltpu

B, N, D = 128, 64, 128
NS = 100
NB = 16
K = 16
RC = 2.0
UPDATE = 0.5
DECAY = 0.9
NCONV = 3
P_OUT = 64
E_OUT = 32

MB = 4
HI = lax.Precision.HIGHEST


CENTERS = (0.5, 0.6000000238418579, 0.7000000476837158, 0.800000011920929,
           0.9000000357627869, 1.0, 1.100000023841858, 1.2000000476837158,
           1.3000000715255737, 1.4000000953674316, 1.5, 1.600000023841858,
           1.7000000476837158, 1.8000000715255737, 1.9000000953674316, 2.0)


def _dot(a, b, precision=None):
    return lax.dot_general(a, b, (((1,), (0,)), ((), ())), precision=precision,
                           preferred_element_type=jnp.float32)


def _mol_kernel(species_ref, coords_ref, ct_ref, emb_ref, wrbf_ref, w1_ref,
                b1_ref, w2_ref, b2_ref, wp1_ref, bp1_ref, wp2_ref, bp2_ref,
                wq_ref, wk_ref, we1_ref, be1_ref, we2_ref, be2_ref,
                iso_ref, an_ref):
    f32 = jnp.float32

    sp3 = species_ref[...]
    oh_t = (sp3 == lax.broadcasted_iota(jnp.int32, (MB, NS, N), 1)).astype(f32)
    emb = emb_ref[...]
    h3 = jnp.stack(
        [lax.dot_general(oh_t[mb], emb, (((0,), (0,)), ((), ())),
                         precision=HI, preferred_element_type=f32)
         for mb in range(MB)], axis=0)
    hf = h3.reshape(MB * N, D)

    coords = coords_ref[...]
    ct = ct_ref[...]
    rx = coords[:, :, 0:1] - ct[:, 0:1, :]
    ry = coords[:, :, 1:2] - ct[:, 1:2, :]
    rz = coords[:, :, 2:3] - ct[:, 2:3, :]
    dist = jnp.sqrt(rx * rx + rz * rz + ry * ry + 1e-12)

    rbf_f = jnp.concatenate(
        [jnp.exp(-10.0 * (dist - np.float32(CENTERS[r])) ** 2)
         for r in range(NB)], axis=1)

    wrbf = wrbf_ref[...]
    w1 = w1_ref[...]; b1 = b1_ref[...]
    w2 = w2_ref[...]; b2 = b2_ref[...]

    for step in range(NCONV):
        h3 = hf.reshape(MB, N, D)
        agg = jnp.stack([_dot(rbf_f[mb], h3[mb]) for mb in range(MB)],
                        axis=0).reshape(MB, NB, N, D)
        ps = [agg[:, r] * wrbf[r:r + 1, :] for r in range(NB)]
        c = [ps[i] + ps[i + 8] for i in range(8)]
        s1 = [c[i] + c[i + 4] for i in range(4)]
        s2 = [s1[i] + s1[i + 2] for i in range(2)]
        m = s2[0] + s2[1]
        mf = m.reshape(MB * N, D)
        upd = _dot(jnp.tanh(_dot(mf, w1) + b1), w2) + b2
        hf = hf + (UPDATE * (DECAY ** step)) * upd

    h3 = hf.reshape(MB, N, D)

    pa = _dot(jnp.tanh(_dot(hf, wp1_ref[...]) + bp1_ref[...]),
              wp2_ref[...]) + bp2_ref[...]
    iso_ref[...] = pa.reshape(MB, N, P_OUT)

    q3 = _dot(hf, wq_ref[...]).reshape(MB, N, D)
    k3 = _dot(hf, wk_ref[...]).reshape(MB, N, D)
    scores = jnp.stack(
        [lax.dot_general(q3[mb], k3[mb], (((1,), (1,)), ((), ())),
                         preferred_element_type=f32)
         for mb in range(MB)], axis=0) / jnp.sqrt(f32(D))

    iota_i = lax.broadcasted_iota(jnp.int32, (MB, N, N), 1)
    iota_j = lax.broadcasted_iota(jnp.int32, (MB, N, N), 2)
    valid = (dist < RC) & (iota_i != iota_j)
    s = jnp.where(valid, scores, f32(-1e9))

    vals = []
    sels = []
    for _ in range(K):
        vmax = jnp.max(s, axis=-1, keepdims=True)
        cand = jnp.where(s == vmax, iota_j, N)
        amin = jnp.min(cand, axis=-1, keepdims=True)
        sel = iota_j == amin
        vals.append(vmax)
        sels.append(sel.astype(f32)[:, :, None, :])
        s = jnp.where(sel, f32(-1e38), s)

    top_vals = jnp.concatenate(vals, axis=-1)
    mx = jnp.max(top_vals, axis=-1, keepdims=True)
    p = jnp.exp(top_vals - mx)
    attn = p / jnp.sum(p, axis=-1, keepdims=True)

    oh2f = jnp.concatenate(sels, axis=2).reshape(MB, N * K, N)
    hsel = jnp.stack([_dot(oh2f[mb], h3[mb], precision=HI) for mb in range(MB)],
                     axis=0).reshape(MB, N, K, D)
    csel = jnp.stack([_dot(oh2f[mb], coords[mb], precision=HI)
                      for mb in range(MB)], axis=0).reshape(MB, N, K, 3)
    rsx = coords[:, :, None, 0:1] - csel[:, :, :, 0:1]
    rsy = coords[:, :, None, 1:2] - csel[:, :, :, 1:2]
    rsz = coords[:, :, None, 2:3] - csel[:, :, :, 2:3]
    dsel = jnp.sqrt(rsx * rsx + rsz * rsz + rsy * rsy + 1e-12)
    den = dsel + 1e-9

    pair = (h3[:, :, None, :] + hsel).reshape(MB * N * K, D)
    e = _dot(jnp.tanh(_dot(pair, we1_ref[...]) + be1_ref[...]),
             we2_ref[...]) + be2_ref[...]
    e4 = e.reshape(MB, N, K, E_OUT)

    ux = rsx / den; uy = rsy / den; uz = rsz / den
    parts = []
    for t in range(K):
        w_t = attn[:, :, t:t + 1][:, :, :, None]
        et = e4[:, :, t:t + 1, :] * w_t
        tt = slice(t, t + 1)
        parts.append(jnp.concatenate(
            [ux[:, :, tt] * et, uy[:, :, tt] * et, uz[:, :, tt] * et],
            axis=-1))
    an_ref[...] = jnp.concatenate(parts, axis=2)


def kernel(species, coords, emb_table, W_rbf, W1, b1, W2, b2, Wp1, bp1, Wp2,
           bp2, Wq, Wk, We1, be1, We2, be2, interpret=False):
    sp = species.astype(jnp.int32).reshape(B, 1, N)
    ct = jnp.swapaxes(coords, 1, 2)
    b1r = b1.reshape(1, D); b2r = b2.reshape(1, D)
    bp1r = bp1.reshape(1, D); bp2r = bp2.reshape(1, P_OUT)
    be1r = be1.reshape(1, D); be2r = be2.reshape(1, E_OUT)

    grid = (B // MB,)
    z2 = lambda i: (0, 0)
    in_specs = [
        pl.BlockSpec((MB, 1, N), lambda i: (i, 0, 0)),
        pl.BlockSpec((MB, N, 3), lambda i: (i, 0, 0)),
        pl.BlockSpec((MB, 3, N), lambda i: (i, 0, 0)),
        pl.BlockSpec((NS, D), z2),
        pl.BlockSpec((NB, D), z2),
        pl.BlockSpec((D, D), z2), pl.BlockSpec((1, D), z2),
        pl.BlockSpec((D, D), z2), pl.BlockSpec((1, D), z2),
        pl.BlockSpec((D, D), z2), pl.BlockSpec((1, D), z2),
        pl.BlockSpec((D, P_OUT), z2), pl.BlockSpec((1, P_OUT), z2),
        pl.BlockSpec((D, D), z2),
        pl.BlockSpec((D, D), z2),
        pl.BlockSpec((D, D), z2), pl.BlockSpec((1, D), z2),
        pl.BlockSpec((D, E_OUT), z2), pl.BlockSpec((1, E_OUT), z2),
    ]
    out_specs = [
        pl.BlockSpec((MB, N, P_OUT), lambda i: (i, 0, 0)),
        pl.BlockSpec((MB, N, K, 3 * E_OUT), lambda i: (i, 0, 0, 0)),
    ]
    out_shape = [
        jax.ShapeDtypeStruct((B, N, P_OUT), jnp.float32),
        jax.ShapeDtypeStruct((B, N, K, 3 * E_OUT), jnp.float32),
    ]
    c_iso, an = pl.pallas_call(
        _mol_kernel,
        grid=grid,
        in_specs=in_specs,
        out_specs=out_specs,
        out_shape=out_shape,
        compiler_params=pltpu.CompilerParams(
            dimension_semantics=("arbitrary",)),
        interpret=interpret,
    )(sp, coords, ct, emb_table, W_rbf, W1, b1r, W2, b2r, Wp1, bp1r, Wp2,
      bp2r, Wq, Wk, We1, be1r, We2, be2r)
    c_aniso = an.reshape(B, N, K * 3, E_OUT)
    return (c_iso, c_aniso)


# speedup vs baseline: 1.2542x; 1.2542x over previous
import numpy as np
import jax
import jax.numpy as jnp
from jax import lax
from jax.experimental import pallas as pl
from jax.experimental.pallas import tpu as pltpu

B, N, D = 128, 64, 128
NS = 100
NB = 16
K = 16
RC = 2.0
UPDATE = 0.5
DECAY = 0.9
NCONV = 3
P_OUT = 64
E_OUT = 32

MB = 4
HI = lax.Precision.HIGHEST


CENTERS = (0.5, 0.6000000238418579, 0.7000000476837158, 0.800000011920929,
           0.9000000357627869, 1.0, 1.100000023841858, 1.2000000476837158,
           1.3000000715255737, 1.4000000953674316, 1.5, 1.600000023841858,
           1.7000000476837158, 1.8000000715255737, 1.9000000953674316, 2.0)


def _dot(a, b, precision=None):
    return lax.dot_general(a, b, (((1,), (0,)), ((), ())), precision=precision,
                           preferred_element_type=jnp.float32)


def _mol_kernel(species_ref, coords_ref, ct_ref, emb_ref, wrbf_ref, w1_ref,
                b1_ref, w2_ref, b2_ref, wp1_ref, bp1_ref, wp2_ref, bp2_ref,
                wq_ref, wk_ref, we1_ref, be1_ref, we2_ref, be2_ref,
                iso_ref, an_ref):
    f32 = jnp.float32

    sp3 = species_ref[...]
    oh_t = (sp3 == lax.broadcasted_iota(jnp.int32, (MB, NS, N), 1)).astype(f32)
    emb = emb_ref[...]
    h3 = jnp.stack(
        [lax.dot_general(oh_t[mb], emb, (((0,), (0,)), ((), ())),
                         precision=HI, preferred_element_type=f32)
         for mb in range(MB)], axis=0)
    hf = h3.reshape(MB * N, D)

    coords = coords_ref[...]
    ct = ct_ref[...]
    rx = coords[:, :, 0:1] - ct[:, 0:1, :]
    ry = coords[:, :, 1:2] - ct[:, 1:2, :]
    rz = coords[:, :, 2:3] - ct[:, 2:3, :]
    dist = jnp.sqrt(rx * rx + rz * rz + ry * ry + 1e-12)

    rbf_f = jnp.concatenate(
        [jnp.exp(-10.0 * (dist - np.float32(CENTERS[r])) ** 2)
         for r in range(NB)], axis=1)

    wrbf = wrbf_ref[...]
    w1 = w1_ref[...]; b1 = b1_ref[...]
    w2 = w2_ref[...]; b2 = b2_ref[...]

    for step in range(NCONV):
        h3 = hf.reshape(MB, N, D)
        agg = jnp.stack([_dot(rbf_f[mb], h3[mb]) for mb in range(MB)],
                        axis=0).reshape(MB, NB, N, D)
        ps = [agg[:, r] * wrbf[r:r + 1, :] for r in range(NB)]
        c = [ps[i] + ps[i + 8] for i in range(8)]
        s1 = [c[i] + c[i + 4] for i in range(4)]
        s2 = [s1[i] + s1[i + 2] for i in range(2)]
        m = s2[0] + s2[1]
        mf = m.reshape(MB * N, D)
        upd = _dot(jnp.tanh(_dot(mf, w1) + b1), w2) + b2
        hf = hf + (UPDATE * (DECAY ** step)) * upd

    h3 = hf.reshape(MB, N, D)

    pa = _dot(jnp.tanh(_dot(hf, wp1_ref[...]) + bp1_ref[...]),
              wp2_ref[...]) + bp2_ref[...]
    iso_ref[...] = pa.reshape(MB, N, P_OUT)

    q3 = _dot(hf, wq_ref[...]).reshape(MB, N, D)
    k3 = _dot(hf, wk_ref[...]).reshape(MB, N, D)
    scores = jnp.stack(
        [lax.dot_general(q3[mb], k3[mb], (((1,), (1,)), ((), ())),
                         preferred_element_type=f32)
         for mb in range(MB)], axis=0) / jnp.sqrt(f32(D))

    iota_i = lax.broadcasted_iota(jnp.int32, (MB, N, N), 1)
    iota_j = lax.broadcasted_iota(jnp.int32, (MB, N, N), 2)
    valid = (dist < RC) & (iota_i != iota_j)
    s = jnp.where(valid, scores, f32(-1e9))

    vals = []
    sels = []
    dsel = []
    rsel = []
    for _ in range(K):
        vmax = jnp.max(s, axis=-1, keepdims=True)
        cand = jnp.where(s == vmax, iota_j, N)
        amin = jnp.min(cand, axis=-1, keepdims=True)
        sel = iota_j == amin
        vals.append(vmax)
        sels.append(sel.astype(f32)[:, :, None, :])
        dsel.append(jnp.sum(jnp.where(sel, dist, 0.0), axis=-1, keepdims=True))
        rsel.append((jnp.sum(jnp.where(sel, rx, 0.0), axis=-1, keepdims=True),
                     jnp.sum(jnp.where(sel, ry, 0.0), axis=-1, keepdims=True),
                     jnp.sum(jnp.where(sel, rz, 0.0), axis=-1, keepdims=True)))
        s = jnp.where(sel, f32(-1e38), s)

    top_vals = jnp.concatenate(vals, axis=-1)
    mx = jnp.max(top_vals, axis=-1, keepdims=True)
    p = jnp.exp(top_vals - mx)
    attn = p / jnp.sum(p, axis=-1, keepdims=True)

    oh2f = jnp.concatenate(sels, axis=2).reshape(MB, N * K, N)
    hsel = jnp.stack([_dot(oh2f[mb], h3[mb], precision=HI) for mb in range(MB)],
                     axis=0).reshape(MB, N, K, D)

    pair = (h3[:, :, None, :] + hsel).reshape(MB * N * K, D)
    e = _dot(jnp.tanh(_dot(pair, we1_ref[...]) + be1_ref[...]),
             we2_ref[...]) + be2_ref[...]
    e4 = e.reshape(MB, N, K, E_OUT)

    uxs, uys, uzs, ats = [], [], [], []
    for t in range(K):
        den = dsel[t] + 1e-9
        sx, sy, sz = rsel[t]
        uxs.append((sx / den)[:, :, :, None])
        uys.append((sy / den)[:, :, :, None])
        uzs.append((sz / den)[:, :, :, None])
        ats.append(attn[:, :, t:t + 1][:, :, :, None])
    ux4 = jnp.concatenate(uxs, axis=2)
    uy4 = jnp.concatenate(uys, axis=2)
    uz4 = jnp.concatenate(uzs, axis=2)
    ea = e4 * jnp.concatenate(ats, axis=2)
    an_ref[...] = jnp.concatenate(
        [ux4 * ea, uy4 * ea, uz4 * ea], axis=-1)


def kernel(species, coords, emb_table, W_rbf, W1, b1, W2, b2, Wp1, bp1, Wp2,
           bp2, Wq, Wk, We1, be1, We2, be2, interpret=False):
    sp = species.astype(jnp.int32).reshape(B, 1, N)
    ct = jnp.swapaxes(coords, 1, 2)
    b1r = b1.reshape(1, D); b2r = b2.reshape(1, D)
    bp1r = bp1.reshape(1, D); bp2r = bp2.reshape(1, P_OUT)
    be1r = be1.reshape(1, D); be2r = be2.reshape(1, E_OUT)

    grid = (B // MB,)
    z2 = lambda i: (0, 0)
    in_specs = [
        pl.BlockSpec((MB, 1, N), lambda i: (i, 0, 0)),
        pl.BlockSpec((MB, N, 3), lambda i: (i, 0, 0)),
        pl.BlockSpec((MB, 3, N), lambda i: (i, 0, 0)),
        pl.BlockSpec((NS, D), z2),
        pl.BlockSpec((NB, D), z2),
        pl.BlockSpec((D, D), z2), pl.BlockSpec((1, D), z2),
        pl.BlockSpec((D, D), z2), pl.BlockSpec((1, D), z2),
        pl.BlockSpec((D, D), z2), pl.BlockSpec((1, D), z2),
        pl.BlockSpec((D, P_OUT), z2), pl.BlockSpec((1, P_OUT), z2),
        pl.BlockSpec((D, D), z2),
        pl.BlockSpec((D, D), z2),
        pl.BlockSpec((D, D), z2), pl.BlockSpec((1, D), z2),
        pl.BlockSpec((D, E_OUT), z2), pl.BlockSpec((1, E_OUT), z2),
    ]
    out_specs = [
        pl.BlockSpec((MB, N, P_OUT), lambda i: (i, 0, 0)),
        pl.BlockSpec((MB, N, K, 3 * E_OUT), lambda i: (i, 0, 0, 0)),
    ]
    out_shape = [
        jax.ShapeDtypeStruct((B, N, P_OUT), jnp.float32),
        jax.ShapeDtypeStruct((B, N, K, 3 * E_OUT), jnp.float32),
    ]
    c_iso, an = pl.pallas_call(
        _mol_kernel,
        grid=grid,
        in_specs=in_specs,
        out_specs=out_specs,
        out_shape=out_shape,
        compiler_params=pltpu.CompilerParams(
            dimension_semantics=("arbitrary",)),
        interpret=interpret,
    )(sp, coords, ct, emb_table, W_rbf, W1, b1r, W2, b2r, Wp1, bp1r, Wp2,
      bp2r, Wq, Wk, We1, be1r, We2, be2r)
    c_aniso = an.reshape(B, N, K * 3, E_OUT)
    return (c_iso, c_aniso)
